# Initial kernel scaffold; baseline (speedup 1.0000x reference)
import jax
import jax.numpy as jnp
from jax.experimental import pallas as pl


def kernel(Yhat):
    raise NotImplementedError("write your pallas kernel here")


# speedup vs baseline: 11.3872x; 11.3872x over previous
import functools

import jax
import jax.numpy as jnp
from jax.experimental import pallas as pl

_BUDGET = 256
_N = 32768
_ROWS_PER_BLOCK = 8
_VAL_ITERS = 34
_IDX_ITERS = 16


def _topk_mask_kernel(y_ref, z_ref):
    y = y_ref[...]
    b = jax.lax.bitcast_convert_type(y + 0.0, jnp.int32)
    s = b ^ jax.lax.shift_right_arithmetic(b, 31).astype(jnp.int32) & jnp.int32(
        0x7FFFFFFF
    )

    k = jnp.int32(_BUDGET)
    r = s.shape[0]
    lo0 = jnp.full((r, 1), jnp.iinfo(jnp.int32).min, jnp.int32)
    hi0 = jnp.full((r, 1), jnp.iinfo(jnp.int32).max, jnp.int32)

    def val_body(_, carry):
        lo, hi = carry
        mid = (lo & hi) + jax.lax.shift_right_arithmetic(lo ^ hi, 1)
        mid = mid + ((lo ^ hi) & 1)
        cnt = jnp.sum((s >= mid).astype(jnp.int32), axis=1, keepdims=True)
        ok = cnt >= k
        return jnp.where(ok, mid, lo), jnp.where(ok, hi, mid - 1)

    v, _ = jax.lax.fori_loop(0, _VAL_ITERS, val_body, (lo0, hi0))

    gt = s > v
    eq = s == v
    need = k - jnp.sum(gt.astype(jnp.int32), axis=1, keepdims=True)

    idx = jax.lax.broadcasted_iota(jnp.int32, s.shape, 1)
    ilo0 = jnp.zeros((r, 1), jnp.int32)
    ihi0 = jnp.full((r, 1), _N - 1, jnp.int32)

    def idx_body(_, carry):
        lo, hi = carry
        mid = jax.lax.shift_right_arithmetic(lo + hi, 1)
        cnt = jnp.sum(
            (eq & (idx <= mid)).astype(jnp.int32), axis=1, keepdims=True
        )
        ok = cnt >= need
        return jnp.where(ok, lo, mid + 1), jnp.where(ok, mid, hi)

    cut, _ = jax.lax.fori_loop(0, _IDX_ITERS, idx_body, (ilo0, ihi0))

    z_ref[...] = (gt | (eq & (idx <= cut))).astype(jnp.float32)


@jax.jit
def kernel(Yhat):
    bsz, n = Yhat.shape
    grid = (bsz // _ROWS_PER_BLOCK,)
    return pl.pallas_call(
        _topk_mask_kernel,
        grid=grid,
        in_specs=[
            pl.BlockSpec((_ROWS_PER_BLOCK, n), lambda i: (i, 0)),
        ],
        out_specs=pl.BlockSpec((_ROWS_PER_BLOCK, n), lambda i: (i, 0)),
        out_shape=jax.ShapeDtypeStruct((bsz, n), jnp.float32),
    )(Yhat)
